# baseline (speedup 1.0000x reference)
import functools

import jax
import jax.numpy as jnp
from jax import lax
from jax.experimental import pallas as pl
from jax.experimental.pallas import tpu as pltpu
from jax.experimental.pallas import tpu_sc as plsc

N_CODES = 8192
D_DIM = 256
BR = 256
_WINDOWS = ((0, 2736), (2736, 5472), (5472, 8192))
_WPAD = 2816


def _argmin_body(zsq_ref, z_ref, e0_ref, e1_ref, e2_ref, idx_ref):
    zb16 = z_ref[...].astype(jnp.bfloat16)
    zsq = zsq_ref[...]
    ii = jax.lax.broadcasted_iota(jnp.int32, (BR, _WPAD), 1)

    rs, idxs = [], []
    for e_ref, (a, b) in zip((e0_ref, e1_ref, e2_ref), _WINDOWS):
        s = jax.lax.dot_general(
            zb16,
            e_ref[...],
            dimension_numbers=(((1,), (0,)), ((), ())),
            preferred_element_type=jnp.float32,
        )
        dw = zsq - 2.0 * s
        r = jnp.min(dw, axis=1, keepdims=True)
        i = jnp.min(jnp.where(dw == r, ii, jnp.int32(N_CODES)), axis=1) + a
        rs.append(r[:, 0])
        idxs.append(i)

    v = rs[0].astype(jnp.bfloat16).astype(jnp.float32)
    idx = idxs[0]
    for k in (1, 2):
        win = rs[k] < v
        idx = jnp.where(win, idxs[k], idx)
        v = jnp.minimum(v, rs[k]).astype(jnp.bfloat16).astype(jnp.float32)

    idx_ref[...] = idx.reshape(1, 1, BR)


def _tc_argmin(zsq, z_flat, e_windows):
    n = z_flat.shape[0]
    nb = n // BR
    ew_spec = pl.BlockSpec((D_DIM, _WPAD), lambda i: (0, 0))
    out = pl.pallas_call(
        _argmin_body,
        grid=(nb,),
        in_specs=[
            pl.BlockSpec((BR, 1), lambda i: (i, 0)),
            pl.BlockSpec((BR, D_DIM), lambda i: (i, 0)),
            ew_spec,
            ew_spec,
            ew_spec,
        ],
        out_specs=pl.BlockSpec((1, 1, BR), lambda i: (i, 0, 0)),
        out_shape=jax.ShapeDtypeStruct((nb, 1, BR), jnp.int32),
        compiler_params=pltpu.CompilerParams(
            dimension_semantics=("parallel",),
        ),
    )(zsq, z_flat, *e_windows)
    return out.reshape(n)


_NW = 32
_CH = 128


def _sc_gather(embedding, idx):
    n = idx.shape[0]
    b_per_w = n // _NW
    mesh = plsc.VectorSubcoreMesh(core_axis_name="c", subcore_axis_name="s")

    @functools.partial(
        pl.kernel,
        mesh=mesh,
        out_type=jax.ShapeDtypeStruct((n, D_DIM), embedding.dtype),
        scratch_types=[
            pltpu.VMEM((b_per_w,), jnp.int32),
            pltpu.VMEM((_CH, D_DIM), jnp.float32),
            pltpu.SemaphoreType.DMA,
        ],
    )
    def k(table_hbm, idx_hbm, out_hbm, idx_v, rows_v, sem):
        wid = lax.axis_index("s") * 2 + lax.axis_index("c")
        base = wid * b_per_w
        pltpu.sync_copy(idx_hbm.at[pl.ds(base, b_per_w)], idx_v)

        @pl.loop(0, b_per_w, step=_CH)
        def _(c):
            pltpu.async_copy(table_hbm.at[idx_v.at[pl.ds(c, _CH)]], rows_v, sem).wait()
            pltpu.sync_copy(rows_v, out_hbm.at[pl.ds(base + c, _CH)])

    return k(embedding, idx)


def kernel(z, embedding):
    B, N, D = z.shape
    z_flat = z.reshape(-1, D)
    zsq = jnp.sum(z_flat ** 2, axis=1, keepdims=True)
    et16 = embedding.T.astype(jnp.bfloat16)
    e_windows = [
        jnp.concatenate(
            [et16[:, a:b], et16[:, a:a + (_WPAD - (b - a))]], axis=1)
        for a, b in _WINDOWS
    ]
    idx = _tc_argmin(zsq, z_flat, e_windows)
    z_q = _sc_gather(embedding, idx)
    z_q = (z_flat + (z_q - z_flat)).reshape(B, N, D)
    return z_q


# speedup vs baseline: 1.0652x; 1.0652x over previous
import functools

import jax
import jax.numpy as jnp
from jax import lax
from jax.experimental import pallas as pl
from jax.experimental.pallas import tpu as pltpu
from jax.experimental.pallas import tpu_sc as plsc

N_CODES = 8192
D_DIM = 256
BR = 512
_WINDOWS = ((0, 2736), (2736, 5472), (5472, 8192))
_WPAD = 2816


def _argmin_body(zsq_ref, z_ref, e0_ref, e1_ref, e2_ref, idx_ref):
    zb16 = z_ref[...].astype(jnp.bfloat16)
    zsq = zsq_ref[...]
    ii = jax.lax.broadcasted_iota(jnp.int32, (BR, _WPAD), 1)

    rs, idxs = [], []
    for e_ref, (a, b) in zip((e0_ref, e1_ref, e2_ref), _WINDOWS):
        s = jax.lax.dot_general(
            zb16,
            e_ref[...],
            dimension_numbers=(((1,), (0,)), ((), ())),
            preferred_element_type=jnp.float32,
        )
        dw = zsq - 2.0 * s
        r = jnp.min(dw, axis=1, keepdims=True)
        i = jnp.min(jnp.where(dw == r, ii, jnp.int32(N_CODES)), axis=1) + a
        rs.append(r[:, 0])
        idxs.append(i)

    v = rs[0].astype(jnp.bfloat16).astype(jnp.float32)
    idx = idxs[0]
    for k in (1, 2):
        win = rs[k] < v
        idx = jnp.where(win, idxs[k], idx)
        v = jnp.minimum(v, rs[k]).astype(jnp.bfloat16).astype(jnp.float32)

    idx_ref[...] = idx.reshape(1, 1, BR)


def _tc_argmin(zsq, z_flat, e_windows):
    n = z_flat.shape[0]
    nb = n // BR
    ew_spec = pl.BlockSpec((D_DIM, _WPAD), lambda i: (0, 0))
    out = pl.pallas_call(
        _argmin_body,
        grid=(nb,),
        in_specs=[
            pl.BlockSpec((BR, 1), lambda i: (i, 0)),
            pl.BlockSpec((BR, D_DIM), lambda i: (i, 0)),
            ew_spec,
            ew_spec,
            ew_spec,
        ],
        out_specs=pl.BlockSpec((1, 1, BR), lambda i: (i, 0, 0)),
        out_shape=jax.ShapeDtypeStruct((nb, 1, BR), jnp.int32),
        compiler_params=pltpu.CompilerParams(
            dimension_semantics=("parallel",),
        ),
    )(zsq, z_flat, *e_windows)
    return out.reshape(n)


_NW = 32
_CH = 128


def _sc_gather(embedding, idx):
    n = idx.shape[0]
    b_per_w = n // _NW
    mesh = plsc.VectorSubcoreMesh(core_axis_name="c", subcore_axis_name="s")

    @functools.partial(
        pl.kernel,
        mesh=mesh,
        out_type=jax.ShapeDtypeStruct((n, D_DIM), embedding.dtype),
        scratch_types=[
            pltpu.VMEM((b_per_w,), jnp.int32),
            pltpu.VMEM((_CH, D_DIM), jnp.float32),
            pltpu.SemaphoreType.DMA,
        ],
    )
    def k(table_hbm, idx_hbm, out_hbm, idx_v, rows_v, sem):
        wid = lax.axis_index("s") * 2 + lax.axis_index("c")
        base = wid * b_per_w
        pltpu.sync_copy(idx_hbm.at[pl.ds(base, b_per_w)], idx_v)

        @pl.loop(0, b_per_w, step=_CH)
        def _(c):
            pltpu.async_copy(table_hbm.at[idx_v.at[pl.ds(c, _CH)]], rows_v, sem).wait()
            pltpu.sync_copy(rows_v, out_hbm.at[pl.ds(base + c, _CH)])

    return k(embedding, idx)


def kernel(z, embedding):
    B, N, D = z.shape
    z_flat = z.reshape(-1, D)
    zsq = jnp.sum(z_flat ** 2, axis=1, keepdims=True)
    et16 = embedding.T.astype(jnp.bfloat16)
    e_windows = [
        jnp.concatenate(
            [et16[:, a:b], et16[:, a:a + (_WPAD - (b - a))]], axis=1)
        for a, b in _WINDOWS
    ]
    idx = _tc_argmin(zsq, z_flat, e_windows)
    z_q = _sc_gather(embedding, idx)
    z_q = (z_flat + (z_q - z_flat)).reshape(B, N, D)
    return z_q
